# Initial kernel scaffold; baseline (speedup 1.0000x reference)
import jax
import jax.numpy as jnp
from jax.experimental import pallas as pl


def kernel(np_batch, table):
    raise NotImplementedError("write your pallas kernel here")


# speedup vs baseline: 1.3461x; 1.3461x over previous
import jax
import jax.numpy as jnp
from jax.experimental import pallas as pl
from jax.experimental.pallas import tpu as pltpu
from jax.experimental.pallas import tpu_sc as plsc

NUM_EMBEDDINGS = 1000000
EMBEDDING_DIM = 32
BATCH = 4096
SEQ_LEN = 200
NUM_IDX = BATCH * SEQ_LEN

WINDOW = 128


def _gather_fn(table, idx_flat):
    vector_mesh = plsc.VectorSubcoreMesh(
        core_axis_name="core", subcore_axis_name="subcore"
    )

    @pl.kernel(
        out_type=jax.ShapeDtypeStruct((NUM_IDX, EMBEDDING_DIM), table.dtype),
        mesh=vector_mesh,
        compiler_params=pltpu.CompilerParams(use_tc_tiling_on_sc=False),
    )
    def kernel_body(x_hbm, i_hbm, o_hbm):
        def body(i_vmem, o_vmem):
            pltpu.sync_copy(x_hbm.at[i_vmem.at[0]], o_vmem)

        pltpu.emit_pipeline(
            body,
            grid=(NUM_IDX // WINDOW,),
            in_specs=[pl.BlockSpec((1, WINDOW), index_map=lambda i: (0, i))],
            out_specs=[
                pl.BlockSpec((WINDOW, EMBEDDING_DIM), index_map=lambda i: (i, 0))
            ],
            core_axis_name=("core", "subcore"),
            dimension_semantics=(pltpu.PARALLEL,),
        )(i_hbm, o_hbm)

    return kernel_body(table, idx_flat)


def kernel(np_batch, table):
    idx_flat = np_batch.astype(jnp.int32).reshape(1, NUM_IDX)
    out = _gather_fn(table, idx_flat)
    return out.reshape(BATCH, SEQ_LEN, EMBEDDING_DIM)


# speedup vs baseline: 1.4684x; 1.0909x over previous
import jax
import jax.numpy as jnp
from jax.experimental import pallas as pl
from jax.experimental.pallas import tpu as pltpu
from jax.experimental.pallas import tpu_sc as plsc

NUM_EMBEDDINGS = 1000000
EMBEDDING_DIM = 32
BATCH = 4096
SEQ_LEN = 200
NUM_IDX = BATCH * SEQ_LEN

WINDOW = 512


def _gather_fn(table, idx_flat):
    vector_mesh = plsc.VectorSubcoreMesh(
        core_axis_name="core", subcore_axis_name="subcore"
    )

    @pl.kernel(
        out_type=jax.ShapeDtypeStruct((NUM_IDX, EMBEDDING_DIM), table.dtype),
        mesh=vector_mesh,
        compiler_params=pltpu.CompilerParams(use_tc_tiling_on_sc=False),
    )
    def kernel_body(x_hbm, i_hbm, o_hbm):
        def body(i_vmem, o_vmem):
            pltpu.sync_copy(x_hbm.at[i_vmem.at[0]], o_vmem)

        pltpu.emit_pipeline(
            body,
            grid=(NUM_IDX // WINDOW,),
            in_specs=[pl.BlockSpec((1, WINDOW), index_map=lambda i: (0, i))],
            out_specs=[
                pl.BlockSpec((WINDOW, EMBEDDING_DIM), index_map=lambda i: (i, 0))
            ],
            core_axis_name=("core", "subcore"),
            dimension_semantics=(pltpu.PARALLEL,),
        )(i_hbm, o_hbm)

    return kernel_body(table, idx_flat)


def kernel(np_batch, table):
    idx_flat = np_batch.astype(jnp.int32).reshape(1, NUM_IDX)
    out = _gather_fn(table, idx_flat)
    return out.reshape(BATCH, SEQ_LEN, EMBEDDING_DIM)


# speedup vs baseline: 1.5406x; 1.0492x over previous
import jax
import jax.numpy as jnp
from jax.experimental import pallas as pl
from jax.experimental.pallas import tpu as pltpu
from jax.experimental.pallas import tpu_sc as plsc

NUM_EMBEDDINGS = 1000000
EMBEDDING_DIM = 32
BATCH = 4096
SEQ_LEN = 200
NUM_IDX = BATCH * SEQ_LEN

WINDOW = 512


def _gather_fn(table, idx_flat):
    vector_mesh = plsc.VectorSubcoreMesh(
        core_axis_name="core", subcore_axis_name="subcore"
    )

    @pl.kernel(
        out_type=jax.ShapeDtypeStruct((NUM_IDX, EMBEDDING_DIM), table.dtype),
        mesh=vector_mesh,
        compiler_params=pltpu.CompilerParams(use_tc_tiling_on_sc=False),
    )
    def kernel_body(x_hbm, i_hbm, o_hbm):
        def body(i_vmem, o_vmem):
            pltpu.sync_copy(x_hbm.at[i_vmem.at[0]], o_vmem)

        pltpu.emit_pipeline(
            body,
            grid=(NUM_IDX // WINDOW,),
            in_specs=[pl.BlockSpec((1, WINDOW), index_map=lambda i: (0, i))],
            out_specs=[
                pl.BlockSpec((WINDOW, EMBEDDING_DIM), index_map=lambda i: (i, 0))
            ],
            core_axis_name=("core", "subcore"),
            dimension_semantics=(pltpu.PARALLEL,),
        )(i_hbm, o_hbm)

    return kernel_body(table, idx_flat)


def kernel(np_batch, table):
    idx_t = jnp.swapaxes(np_batch, 0, 1).astype(jnp.int32).reshape(1, NUM_IDX)
    out_t = _gather_fn(table, idx_t)
    out_t = out_t.reshape(SEQ_LEN, BATCH, EMBEDDING_DIM)
    return jnp.transpose(out_t, (1, 0, 2))
